# baseline (speedup 1.0000x reference)
import functools

import jax
import jax.numpy as jnp
from jax import lax
from jax.experimental import pallas as pl
from jax.experimental.pallas import tpu as pltpu
from jax.experimental.pallas import tpu_sc as plsc

N_NODES = 10000
N_EDGES = 320000
D = 128

NC = 2
NS = 16
L = 16
NW = NC * NS

EDGES_PER_TILE = N_EDGES // NW
CHUNK = 80
N_CHUNKS = EDGES_PER_TILE // CHUNK
NR = 3
NE = 6
WIDE = 6
ROWS_MAIN = 624
TAIL0 = NS * ROWS_MAIN
TAIL = N_NODES - TAIL0
ZROWS = 16

_mesh = plsc.VectorSubcoreMesh(core_axis_name="c", subcore_axis_name="s")


@functools.partial(
    pl.kernel,
    mesh=_mesh,
    compiler_params=pltpu.CompilerParams(needs_layout_passes=False, use_tc_tiling_on_sc=False),
    out_type=jax.ShapeDtypeStruct((NC, N_NODES, D), jnp.float32),
    scratch_types=(
        [pltpu.VMEM((1, CHUNK), jnp.int32) for _ in range(NE)]
        + [pltpu.VMEM((1, CHUNK), jnp.int32) for _ in range(NE)]
        + [pltpu.VMEM((1, CHUNK), jnp.float32) for _ in range(NE)]
        + [pltpu.VMEM((CHUNK, D // 2), jnp.int32) for _ in range(NR)]
        + [pltpu.VMEM((CHUNK, D), jnp.float32) for _ in range(NR)]
        + [pltpu.VMEM((ZROWS, D), jnp.float32),
           pltpu.VMEM_SHARED((N_NODES, D), jnp.float32)]
        + [pltpu.SemaphoreType.DMA] * (NE + 2 * NR + 1)
    ),
)
def _sc_aggregate(x_hbm, es_hbm, ed_hbm, ew_hbm, out_hbm, *refs):
    sbufs = refs[0:NE]
    dbufs = refs[NE:2 * NE]
    wbufs = refs[2 * NE:3 * NE]
    rbufs = refs[3 * NE:3 * NE + NR]
    obufs = refs[3 * NE + NR:3 * NE + 2 * NR]
    zero_v = refs[3 * NE + 2 * NR]
    acc = refs[3 * NE + 2 * NR + 1]
    isems = refs[3 * NE + 2 * NR + 2:4 * NE + 2 * NR + 2]
    gsems = refs[4 * NE + 2 * NR + 2:4 * NE + 3 * NR + 2]
    ssems = refs[4 * NE + 3 * NR + 2:4 * NE + 4 * NR + 2]
    zsem = refs[4 * NE + 4 * NR + 2]

    cid = lax.axis_index("c")
    sid = lax.axis_index("s")
    wid = cid * NS + sid

    def load_edata(i, b):
        eb = b % NE
        sl = pl.ds(i, 1)
        pltpu.async_copy(es_hbm.at[wid, sl, :], sbufs[eb], isems[eb])
        pltpu.async_copy(ed_hbm.at[wid, sl, :], dbufs[eb], isems[eb])
        pltpu.async_copy(ew_hbm.at[wid, sl, :], wbufs[eb], isems[eb])

    def wait_edata(b):
        eb = b % NE
        sl = pl.ds(0, 1)
        pltpu.make_async_copy(es_hbm.at[wid, sl, :], sbufs[eb],
                              isems[eb]).wait()
        pltpu.make_async_copy(ed_hbm.at[wid, sl, :], dbufs[eb],
                              isems[eb]).wait()
        pltpu.make_async_copy(ew_hbm.at[wid, sl, :], wbufs[eb],
                              isems[eb]).wait()

    def start_gather(b):
        pltpu.async_copy(x_hbm.at[sbufs[b % NE].at[0]], rbufs[b % NR],
                         gsems[b % NR])

    def wait_gather(b):
        pltpu.make_async_copy(x_hbm.at[sbufs[b % NE].at[0]], rbufs[b % NR],
                              gsems[b % NR]).wait()

    def start_scatter(b):
        pltpu.async_copy(obufs[b % NR], acc.at[dbufs[b % NE].at[0]],
                         ssems[b % NR], add=True)

    def wait_scatter(b):
        pltpu.make_async_copy(obufs[b % NR], acc.at[dbufs[b % NE].at[0]],
                              ssems[b % NR]).wait()

    def scale(b):
        rb, ob, wbuf = rbufs[b % NR], obufs[b % NR], wbufs[b % NE]

        def grp(g, carry):
            wv = wbuf[0, pl.ds(g * L, L)]
            for j in range(L):
                wb = jnp.broadcast_to(wv[j], (L,))
                e = g * L + j
                for m in range(D // (2 * L)):
                    vi = rb[e, pl.ds(m * L, L)]
                    v32 = plsc.bitcast(vi, jnp.bfloat16)
                    a, bb = plsc.unpack(v32, format=plsc.PackFormat.INTERLEAVED)
                    ob[e, pl.ds(m * 2 * L, L)] = a * wb
                    ob[e, pl.ds(m * 2 * L + L, L)] = bb * wb
            return carry

        lax.fori_loop(0, CHUNK // L, grp, 0, unroll=1)

    def chunk_step(i, b, skip_free=False, skip_load=False, skip_gather=False):
        if not skip_free:
            wait_scatter(b - 2)
        if not skip_load:
            load_edata(i + 4, b + 4)
        if not skip_gather:
            wait_edata(b + 2)
            start_gather(b + 2)
        wait_gather(b)
        scale(b)
        start_scatter(b)

    for j in range(4):
        load_edata(j, j)

    zvec = jnp.zeros((L,), jnp.float32)

    def zrow(r, carry):
        for v in range(D // L):
            zero_v[r, pl.ds(v * L, L)] = zvec
        return carry

    lax.fori_loop(0, ZROWS, zrow, 0)
    row0 = pl.multiple_of(sid * ROWS_MAIN, 8)
    zslices = [pl.ds(row0 + k * ZROWS, ZROWS)
               for k in range(ROWS_MAIN // ZROWS)]
    for sl in zslices:
        pltpu.async_copy(zero_v, acc.at[sl, :], zsem)

    @pl.when(sid == NS - 1)
    def _zero_tail():
        pltpu.sync_copy(zero_v.at[pl.ds(0, TAIL), :],
                        acc.at[pl.ds(TAIL0, TAIL), :])

    wait_edata(0)
    start_gather(0)
    wait_edata(1)
    start_gather(1)
    for sl in zslices:
        pltpu.make_async_copy(zero_v, acc.at[sl, :], zsem).wait()
    plsc.subcore_barrier()

    for i in range(WIDE):
        chunk_step(i, i, skip_free=(i < 2))

    def sextet(t, carry):
        i0 = WIDE * t
        for b in range(WIDE):
            chunk_step(i0 + b, b)
        return carry

    lax.fori_loop(1, 120 // WIDE, sextet, 0)

    for i in range(120, N_CHUNKS):
        chunk_step(i, i, skip_load=(i + 4 > N_CHUNKS - 1),
                   skip_gather=(i + 2 > N_CHUNKS - 1))

    wait_scatter(N_CHUNKS - 2)
    wait_scatter(N_CHUNKS - 1)
    plsc.subcore_barrier()

    pltpu.sync_copy(acc.at[pl.ds(row0, ROWS_MAIN), :],
                    out_hbm.at[cid, pl.ds(row0, ROWS_MAIN), :])

    @pl.when(sid == NS - 1)
    def _write_tail():
        pltpu.sync_copy(acc.at[pl.ds(TAIL0, TAIL), :],
                        out_hbm.at[cid, pl.ds(TAIL0, TAIL), :])


BM = 1000


def _tc_body(p_ref, w_ref, o_ref):
    s = p_ref[0] + p_ref[1]
    o_ref[...] = jnp.maximum(
        jnp.dot(s, w_ref[...], preferred_element_type=jnp.float32), 0.0)


def _tc_combine(partials, W):
    return pl.pallas_call(
        _tc_body,
        grid=(N_NODES // BM,),
        in_specs=[
            pl.BlockSpec((NC, BM, D), lambda i: (0, i, 0)),
            pl.BlockSpec((D, D), lambda i: (0, 0)),
        ],
        out_specs=pl.BlockSpec((BM, D), lambda i: (i, 0)),
        out_shape=jax.ShapeDtypeStruct((N_NODES, D), jnp.float32),
    )(partials, W)


def kernel(x, edge_index, edge_weight, W):
    src = edge_index[1].astype(jnp.int32).reshape(NW, N_CHUNKS, CHUNK)
    dst = edge_index[0].astype(jnp.int32).reshape(NW, N_CHUNKS, CHUNK)
    ew = edge_weight.reshape(NW, N_CHUNKS, CHUNK)
    xi = x.reshape(N_NODES, D // 32, 2, L).swapaxes(2, 3).reshape(N_NODES, D)
    xb = xi.astype(jnp.bfloat16).reshape(N_NODES, D // 2, 2)
    xp = jax.lax.bitcast_convert_type(xb, jnp.int32)
    partials = _sc_aggregate(xp, src, dst, ew)
    return _tc_combine(partials, W)


# speedup vs baseline: 1.9213x; 1.9213x over previous
import functools

import jax
import jax.numpy as jnp
from jax import lax
from jax.experimental import pallas as pl
from jax.experimental.pallas import tpu as pltpu
from jax.experimental.pallas import tpu_sc as plsc

N_NODES = 10000
N_EDGES = 320000
D = 128

NC = 2
NS = 16
L = 16
NW = NC * NS

EDGES_PER_TILE = N_EDGES // NW
CHUNK = 80
N_CHUNKS = EDGES_PER_TILE // CHUNK
NR = 4
NE = 8
ROWS_MAIN = 624
TAIL0 = NS * ROWS_MAIN
TAIL = N_NODES - TAIL0
ZROWS = 48

_mesh = plsc.VectorSubcoreMesh(core_axis_name="c", subcore_axis_name="s")


@functools.partial(
    pl.kernel,
    mesh=_mesh,
    out_type=jax.ShapeDtypeStruct((NC, N_NODES, D), jnp.float32),
    scratch_types=(
        [pltpu.VMEM((1, CHUNK), jnp.int32) for _ in range(NE)]
        + [pltpu.VMEM((1, CHUNK), jnp.int32) for _ in range(NE)]
        + [pltpu.VMEM((1, CHUNK), jnp.float32) for _ in range(NE)]
        + [pltpu.VMEM((CHUNK, D), jnp.float32) for _ in range(NR)]
        + [pltpu.VMEM((ZROWS, D), jnp.float32),
           pltpu.VMEM_SHARED((N_NODES, D), jnp.float32)]
        + [pltpu.SemaphoreType.DMA] * (NE + 2 * NR + 1)
    ),
)
def _sc_aggregate(x_hbm, es_hbm, ed_hbm, ew_hbm, out_hbm, *refs):
    sbufs = refs[0:NE]
    dbufs = refs[NE:2 * NE]
    wbufs = refs[2 * NE:3 * NE]
    rbufs = refs[3 * NE:3 * NE + NR]
    zero_v = refs[3 * NE + NR]
    acc = refs[3 * NE + NR + 1]
    isems = refs[3 * NE + NR + 2:4 * NE + NR + 2]
    gsems = refs[4 * NE + NR + 2:4 * NE + 2 * NR + 2]
    ssems = refs[4 * NE + 2 * NR + 2:4 * NE + 3 * NR + 2]
    zsem = refs[4 * NE + 3 * NR + 2]

    cid = lax.axis_index("c")
    sid = lax.axis_index("s")
    wid = cid * NS + sid

    def load_edata(i, b):
        eb = b % NE
        sl = pl.ds(i, 1)
        pltpu.async_copy(es_hbm.at[wid, sl, :], sbufs[eb], isems[eb])
        pltpu.async_copy(ed_hbm.at[wid, sl, :], dbufs[eb], isems[eb])
        pltpu.async_copy(ew_hbm.at[wid, sl, :], wbufs[eb], isems[eb])

    def wait_edata(b):
        eb = b % NE
        sl = pl.ds(0, 1)
        pltpu.make_async_copy(es_hbm.at[wid, sl, :], sbufs[eb],
                              isems[eb]).wait()
        pltpu.make_async_copy(ed_hbm.at[wid, sl, :], dbufs[eb],
                              isems[eb]).wait()
        pltpu.make_async_copy(ew_hbm.at[wid, sl, :], wbufs[eb],
                              isems[eb]).wait()

    def start_gather(b):
        pltpu.async_copy(x_hbm.at[sbufs[b % NE].at[0]], rbufs[b % NR],
                         gsems[b % NR])

    def wait_gather(b):
        pltpu.make_async_copy(x_hbm.at[sbufs[b % NE].at[0]], rbufs[b % NR],
                              gsems[b % NR]).wait()

    def start_scatter(b):
        pltpu.async_copy(rbufs[b % NR], acc.at[dbufs[b % NE].at[0]],
                         ssems[b % NR], add=True)

    def wait_scatter(b):
        pltpu.make_async_copy(rbufs[b % NR], acc.at[dbufs[b % NE].at[0]],
                              ssems[b % NR]).wait()

    def scale(b):
        rb, wbuf = rbufs[b % NR], wbufs[b % NE]

        def grp(g, carry):
            wv = wbuf[0, pl.ds(g * L, L)]
            for j in range(L):
                wb = jnp.broadcast_to(wv[j], (L,))
                e = g * L + j
                for v in range(D // L):
                    rb[e, pl.ds(v * L, L)] = rb[e, pl.ds(v * L, L)] * wb
            return carry

        lax.fori_loop(0, CHUNK // L, grp, 0, unroll=1)

    def chunk_step(i, b, skip_free=False, skip_load=False, skip_gather=False):
        if not skip_load:
            load_edata(i + 4, b + 4)
        if not skip_free:
            wait_scatter(b - 2)
        if not skip_gather:
            wait_edata(b + 2)
            start_gather(b + 2)
        wait_gather(b)
        scale(b)
        start_scatter(b)

    for j in range(4):
        load_edata(j, j)

    zvec = jnp.zeros((L,), jnp.float32)

    def zrow(r, carry):
        for v in range(D // L):
            zero_v[r, pl.ds(v * L, L)] = zvec
        return carry

    lax.fori_loop(0, ZROWS, zrow, 0)
    row0 = pl.multiple_of(sid * ROWS_MAIN, 8)
    zcopies = [(pl.ds(row0 + k * ZROWS, ZROWS), ZROWS)
               for k in range(ROWS_MAIN // ZROWS)]
    for sl, _ in zcopies:
        pltpu.async_copy(zero_v, acc.at[sl, :], zsem)

    @pl.when(sid == NS - 1)
    def _zero_tail():
        pltpu.sync_copy(zero_v.at[pl.ds(0, TAIL), :],
                        acc.at[pl.ds(TAIL0, TAIL), :])

    wait_edata(0)
    start_gather(0)
    wait_edata(1)
    start_gather(1)
    for sl, _ in zcopies:
        pltpu.make_async_copy(zero_v, acc.at[sl, :], zsem).wait()
    plsc.subcore_barrier()

    for i in range(8):
        chunk_step(i, i, skip_free=(i < 2))

    def octet(t, carry):
        i0 = 8 * t
        for b in range(8):
            chunk_step(i0 + b, b)
        return carry

    lax.fori_loop(1, N_CHUNKS // 8, octet, 0)

    for i in range(120, N_CHUNKS):
        chunk_step(i, i, skip_load=(i + 4 > N_CHUNKS - 1),
                   skip_gather=(i + 2 > N_CHUNKS - 1))

    wait_scatter(N_CHUNKS - 2)
    wait_scatter(N_CHUNKS - 1)
    plsc.subcore_barrier()

    pltpu.sync_copy(acc.at[pl.ds(row0, ROWS_MAIN), :],
                    out_hbm.at[cid, pl.ds(row0, ROWS_MAIN), :])

    @pl.when(sid == NS - 1)
    def _write_tail():
        pltpu.sync_copy(acc.at[pl.ds(TAIL0, TAIL), :],
                        out_hbm.at[cid, pl.ds(TAIL0, TAIL), :])


BM = 1000


def _tc_body(p_ref, w_ref, o_ref):
    s = p_ref[0] + p_ref[1]
    o_ref[...] = jnp.maximum(
        jnp.dot(s, w_ref[...], preferred_element_type=jnp.float32), 0.0)


def _tc_combine(partials, W):
    return pl.pallas_call(
        _tc_body,
        grid=(N_NODES // BM,),
        in_specs=[
            pl.BlockSpec((NC, BM, D), lambda i: (0, i, 0)),
            pl.BlockSpec((D, D), lambda i: (0, 0)),
        ],
        out_specs=pl.BlockSpec((BM, D), lambda i: (i, 0)),
        out_shape=jax.ShapeDtypeStruct((N_NODES, D), jnp.float32),
    )(partials, W)


def kernel(x, edge_index, edge_weight, W):
    src = edge_index[1].astype(jnp.int32).reshape(NW, N_CHUNKS, CHUNK)
    dst = edge_index[0].astype(jnp.int32).reshape(NW, N_CHUNKS, CHUNK)
    ew = edge_weight.reshape(NW, N_CHUNKS, CHUNK)
    partials = _sc_aggregate(x, src, dst, ew)
    return _tc_combine(partials, W)
